# baseline (speedup 1.0000x reference)
import functools

import jax
import jax.numpy as jnp
from jax import lax
from jax.experimental import pallas as pl
from jax.experimental.pallas import tpu as pltpu
from jax.experimental.pallas import tpu_sc as plsc

N = 50000
NPAD = 51200
E = 800000
IN_CH = 6
HID = 64
HALF = HID // 2
G = 64
NB = 2048
NBP = NB // 4
GRID = NPAD // NB
NPK = NPAD // 4

NCORE = 2
NSUB = 16
ROWS_PER_TILE = NPAD // NSUB
CH = 128
NCHUNK = E // CH
MP_BASE = NCHUNK // NSUB
MP_EXTRA = NCHUNK % NSUB
DG_BASE = NCHUNK // (NCORE * NSUB)
DG_EXTRA = NCHUNK % (NCORE * NSUB)
STAGE_ROWS = 200
STAGE_ITERS = ROWS_PER_TILE // STAGE_ROWS

_F32 = jnp.float32
_PREC = jax.lax.Precision.HIGHEST


def _sc_mesh():
    return plsc.VectorSubcoreMesh(core_axis_name="c", subcore_axis_name="s")


def _deg_body(dstd_hbm, zrow_hbm, deg_out, idxd, ones_v, stage_v, acc,
              sem_i, sem_s):
    c = lax.axis_index("c")
    s = lax.axis_index("s")
    for k in range(CH // 16):
        ones_v[pl.ds(k * 16, 16)] = jnp.ones((16,), _F32)
    nbase = s * ROWS_PER_TILE
    pltpu.sync_copy(zrow_hbm, stage_v)
    pltpu.sync_copy(stage_v, acc.at[pl.ds(nbase, ROWS_PER_TILE)])
    plsc.subcore_barrier()
    t = c * NSUB + s
    rbase = DG_BASE * t + jnp.minimum(t, DG_EXTRA)
    nch = DG_BASE + jnp.where(t < DG_EXTRA, 1, 0)

    def idx_copy(i):
        return pltpu.make_async_copy(dstd_hbm.at[rbase + i],
                                     idxd.at[lax.rem(i, 3)], sem_i)

    def sc_start(i):
        pltpu.async_copy(ones_v, acc.at[idxd.at[lax.rem(i, 3)]], sem_s,
                         add=True)

    def sc_wait(i):
        pltpu.make_async_copy(ones_v, acc.at[idxd.at[lax.rem(i, 3)]],
                              sem_s).wait()

    idx_copy(0).start()
    idx_copy(1).start()

    def step(i, carry):
        idx_copy(i).wait()

        @pl.when(i > 0)
        def _():
            sc_wait(i - 1)

        @pl.when(i + 2 < nch)
        def _():
            idx_copy(i + 2).start()

        sc_start(i)
        return carry

    lax.fori_loop(0, nch, step, 0)
    sc_wait(nch - 1)
    plsc.subcore_barrier()
    pltpu.sync_copy(acc.at[pl.ds(nbase, ROWS_PER_TILE)], stage_v)
    pltpu.sync_copy(stage_v,
                    deg_out.at[pl.ds(c * NPAD + nbase, ROWS_PER_TILE)])


def _deg_call(dstd, zrow1d):
    return pl.kernel(
        _deg_body,
        out_type=jax.ShapeDtypeStruct((NCORE * NPAD,), _F32),
        mesh=_sc_mesh(),
        scratch_types=[
            pltpu.VMEM((3, CH), jnp.int32),
            pltpu.VMEM((CH,), _F32),
            pltpu.VMEM((ROWS_PER_TILE,), _F32),
            pltpu.VMEM_SHARED((NPAD,), _F32),
            pltpu.SemaphoreType.DMA,
            pltpu.SemaphoreType.DMA,
        ],
        compiler_params=pltpu.CompilerParams(use_tc_tiling_on_sc=False),
    )(dstd, zrow1d)


def _mp_body(y0_hbm, y1_hbm, pidx_hbm, zrows_hbm, s0_out, s1_out,
             pidxv, rows_v, stage_v, acc, sem_i, sem_g, sem_s):
    c = lax.axis_index("c")
    s = lax.axis_index("s")
    nbase = s * ROWS_PER_TILE
    pltpu.sync_copy(zrows_hbm, stage_v)

    def zinit(k, carry):
        pltpu.sync_copy(stage_v, acc.at[pl.ds(nbase + k * STAGE_ROWS,
                                              STAGE_ROWS)])
        return carry

    lax.fori_loop(0, STAGE_ITERS, zinit, 0)
    plsc.subcore_barrier()
    rbase = MP_BASE * s + jnp.minimum(s, MP_EXTRA)
    nch = MP_BASE + jnp.where(s < MP_EXTRA, 1, 0)

    def run(y32, s_out):
        def idx_copy(i):
            return pltpu.make_async_copy(
                pidx_hbm.at[pl.ds(2 * (rbase + i), 2)],
                pidxv.at[lax.rem(i, 3)], sem_i)

        def g_copy(i):
            return pltpu.make_async_copy(y32.at[pidxv.at[lax.rem(i, 3), 0]],
                                         rows_v.at[lax.rem(i, 2)], sem_g)

        def s_start(i):
            pltpu.async_copy(rows_v.at[lax.rem(i, 2)],
                             acc.at[pidxv.at[lax.rem(i, 3), 1]], sem_s,
                             add=True)

        def s_wait(i):
            pltpu.make_async_copy(rows_v.at[lax.rem(i, 2)],
                                  acc.at[pidxv.at[lax.rem(i, 3), 1]],
                                  sem_s).wait()

        idx_copy(0).start()
        idx_copy(1).start()
        idx_copy(0).wait()
        g_copy(0).start()

        def step(i, carry):
            @pl.when(i + 1 < nch)
            def _():
                idx_copy(i + 1).wait()

            @pl.when(i > 0)
            def _():
                s_wait(i - 1)

            @pl.when(i + 2 < nch)
            def _():
                idx_copy(i + 2).start()

            g_copy(i).wait()

            @pl.when(i + 1 < nch)
            def _():
                g_copy(i + 1).start()

            s_start(i)
            return carry

        lax.fori_loop(0, nch, step, 0)
        s_wait(nch - 1)
        plsc.subcore_barrier()
        out32 = s_out

        def copyout(k, carry):
            rb = nbase + k * STAGE_ROWS
            pltpu.sync_copy(acc.at[pl.ds(rb, STAGE_ROWS)], stage_v)
            pltpu.sync_copy(stage_v, out32.at[pl.ds(rb, STAGE_ROWS)])
            return carry

        lax.fori_loop(0, STAGE_ITERS, copyout, 0)

    pl.when(c == 0)(lambda: run(y0_hbm, s0_out))
    pl.when(c == 1)(lambda: run(y1_hbm, s1_out))


def _mp_call(y0, y1, pidx, zrows):
    return pl.kernel(
        _mp_body,
        out_type=[jax.ShapeDtypeStruct((NPAD, HALF), _F32),
                  jax.ShapeDtypeStruct((NPAD, HALF), _F32)],
        mesh=_sc_mesh(),
        scratch_types=[
            pltpu.VMEM((3, 2, CH), jnp.int32),
            pltpu.VMEM((2, CH, HALF), _F32),
            pltpu.VMEM((STAGE_ROWS, HALF), _F32),
            pltpu.VMEM_SHARED((NPAD, HALF), _F32),
            pltpu.SemaphoreType.DMA,
            pltpu.SemaphoreType.DMA,
            pltpu.SemaphoreType.DMA,
        ],
        compiler_params=pltpu.CompilerParams(use_tc_tiling_on_sc=False),
    )(y0, y1, pidx, zrows)


def _bdiag(q, nrep=4):
    r, c = q.shape
    z = jnp.zeros((r, c), _F32)
    rows = [jnp.concatenate([q if j == i else z for j in range(nrep)], axis=1)
            for i in range(nrep)]
    return jnp.concatenate(rows, axis=0)


def _rep4():
    return (lax.broadcasted_iota(jnp.int32, (4, 128), 1) // HALF ==
            lax.broadcasted_iota(jnp.int32, (4, 128), 0)).astype(_F32)


def _bias_pack(b_ref, half):
    row = b_ref[...][:, half * HALF:(half + 1) * HALF]
    return jnp.concatenate([row] * 4, axis=1)


def _dot(a, b):
    return jnp.dot(a, b, preferred_element_type=_F32, precision=_PREC)


def _prep1_body(deg4_ref, x4_ref, w1_ref, y0_ref, y1_ref, dinvp_ref):
    deg4 = deg4_ref[0] + deg4_ref[1] + 1.0
    dinvp = _dot(lax.rsqrt(deg4), _rep4())
    w1 = w1_ref[...]
    xw0p = _dot(x4_ref[...], _bdiag(w1[:, :HALF]))
    xw1p = _dot(x4_ref[...], _bdiag(w1[:, HALF:]))
    y0_ref[...] = xw0p * dinvp
    y1_ref[...] = xw1p * dinvp
    dinvp_ref[...] = dinvp


def _prep1_call(deg4, x4, w1):
    return pl.pallas_call(
        _prep1_body,
        grid=(GRID,),
        in_specs=[
            pl.BlockSpec((NCORE, NBP, 4), lambda i: (0, i, 0)),
            pl.BlockSpec((NBP, 4 * IN_CH), lambda i: (i, 0)),
            pl.BlockSpec((IN_CH, HID), lambda i: (0, 0)),
        ],
        out_specs=[
            pl.BlockSpec((NBP, 128), lambda i: (i, 0)),
            pl.BlockSpec((NBP, 128), lambda i: (i, 0)),
            pl.BlockSpec((NBP, 128), lambda i: (i, 0)),
        ],
        out_shape=[
            jax.ShapeDtypeStruct((NPK, 128), _F32),
            jax.ShapeDtypeStruct((NPK, 128), _F32),
            jax.ShapeDtypeStruct((NPK, 128), _F32),
        ],
    )(deg4, x4, w1)


def _prep2_body(s0_ref, s1_ref, y0_ref, y1_ref, dinvp_ref, w2_ref, b1_ref,
                y20_ref, y21_ref):
    dinvp = dinvp_ref[...]
    h0p = jnp.maximum((s0_ref[...] + y0_ref[...]) * dinvp
                      + _bias_pack(b1_ref, 0), 0.0)
    h1p = jnp.maximum((s1_ref[...] + y1_ref[...]) * dinvp
                      + _bias_pack(b1_ref, 1), 0.0)
    w2 = w2_ref[...]
    y20_ref[...] = (_dot(h0p, _bdiag(w2[:HALF, :HALF])) +
                    _dot(h1p, _bdiag(w2[HALF:, :HALF]))) * dinvp
    y21_ref[...] = (_dot(h0p, _bdiag(w2[:HALF, HALF:])) +
                    _dot(h1p, _bdiag(w2[HALF:, HALF:]))) * dinvp


def _prep2_call(s0, s1, y0, y1, dinvp, w2, b1):
    blk = pl.BlockSpec((NBP, 128), lambda i: (i, 0))
    return pl.pallas_call(
        _prep2_body,
        grid=(GRID,),
        in_specs=[blk, blk, blk, blk, blk,
                  pl.BlockSpec((HID, HID), lambda i: (0, 0)),
                  pl.BlockSpec((1, HID), lambda i: (0, 0))],
        out_specs=[blk, blk],
        out_shape=[
            jax.ShapeDtypeStruct((NPK, 128), _F32),
            jax.ShapeDtypeStruct((NPK, 128), _F32),
        ],
    )(s0, s1, y0, y1, dinvp, w2, b1)


def _pool_body(s0_ref, s1_ref, y20_ref, y21_ref, dinvp_ref, b2_ref,
               batch_ref, out_ref, acc_ref):
    i = pl.program_id(0)
    dinvp = dinvp_ref[...]
    h0p = (s0_ref[...] + y20_ref[...]) * dinvp + _bias_pack(b2_ref, 0)
    h1p = (s1_ref[...] + y21_ref[...]) * dinvp + _bias_pack(b2_ref, 1)
    seg0 = jnp.zeros((G, HALF), _F32)
    seg1 = jnp.zeros((G, HALF), _F32)
    counts = jnp.zeros((G, 1), _F32)
    giota = lax.broadcasted_iota(jnp.int32, (G, NBP), 0)
    b4 = batch_ref[0]
    for a in range(4):
        ohp = (giota == b4[a:a + 1, :]).astype(_F32)
        seg0 = seg0 + _dot(ohp, h0p[:, a * HALF:(a + 1) * HALF])
        seg1 = seg1 + _dot(ohp, h1p[:, a * HALF:(a + 1) * HALF])
        counts = counts + _dot(ohp, jnp.ones((NBP, 1), _F32))
    p = jnp.concatenate([seg0, seg1, counts,
                         jnp.zeros((G, HID - 1), _F32)], axis=1)

    @pl.when(i == 0)
    def _():
        acc_ref[...] = p

    @pl.when(i > 0)
    def _():
        acc_ref[...] += p

    @pl.when(i == GRID - 1)
    def _():
        a_ = acc_ref[...]
        cnt = a_[:, HID:HID + 1]
        out_ref[...] = a_[:, :HID] / jnp.maximum(cnt, 1.0)


def _pool_call(s0, s1, y20, y21, dinvp, b2, batch4):
    blk = pl.BlockSpec((NBP, 128), lambda i: (i, 0))
    return pl.pallas_call(
        _pool_body,
        grid=(GRID,),
        in_specs=[blk, blk, blk, blk, blk,
                  pl.BlockSpec((1, HID), lambda i: (0, 0)),
                  pl.BlockSpec((1, 4, NBP), lambda i: (i, 0, 0))],
        out_specs=pl.BlockSpec((G, HID), lambda i: (0, 0)),
        out_shape=jax.ShapeDtypeStruct((G, HID), _F32),
        scratch_shapes=[pltpu.VMEM((G, 2 * HID), _F32)],
    )(s0, s1, y20, y21, dinvp, b2, batch4)


@functools.partial(jax.jit)
def kernel(x, edge_index, edge_attr, batch, W1, b1, W2, b2):
    del edge_attr
    src = edge_index[0].astype(jnp.int32)
    dst = edge_index[1].astype(jnp.int32)
    x4 = jnp.pad(x, ((0, NPAD - N), (0, 0))).reshape(NPK, 4 * IN_CH)
    batch4 = jnp.pad(batch.astype(jnp.int32), (0, NPAD - N),
                     constant_values=G
                     ).reshape(GRID, NBP, 4).transpose(0, 2, 1)
    zrow1d = jnp.zeros((ROWS_PER_TILE,), _F32)
    zrows = jnp.zeros((STAGE_ROWS, HALF), _F32)
    b1r = b1.reshape(1, HID)
    b2r = b2.reshape(1, HID)
    pidx = jnp.stack([src.reshape(NCHUNK, CH), dst.reshape(NCHUNK, CH)],
                     axis=1).reshape(2 * NCHUNK, CH)
    dstd = dst.reshape(NCHUNK, CH)

    degp = _deg_call(dstd, zrow1d)
    y0, y1, dinvp = _prep1_call(degp.reshape(NCORE, NPK, 4), x4, W1)
    s0, s1 = _mp_call(y0.reshape(NPAD, HALF), y1.reshape(NPAD, HALF),
                      pidx, zrows)
    y20, y21 = _prep2_call(s0.reshape(NPK, 128), s1.reshape(NPK, 128),
                           y0, y1, dinvp, W2, b1r)
    t0, t1 = _mp_call(y20.reshape(NPAD, HALF), y21.reshape(NPAD, HALF),
                      pidx, zrows)
    return _pool_call(t0.reshape(NPK, 128), t1.reshape(NPK, 128),
                      y20, y21, dinvp, b2r, batch4)


# speedup vs baseline: 1.6784x; 1.6784x over previous
import functools

import jax
import jax.numpy as jnp
from jax import lax
from jax.experimental import pallas as pl
from jax.experimental.pallas import tpu as pltpu
from jax.experimental.pallas import tpu_sc as plsc

N = 50000
NPAD = 51200
E = 800000
IN_CH = 6
HID = 64
HALF = HID // 2
G = 64
NB = 2048
NBP = NB // 4
GRID = NPAD // NB
NPK = NPAD // 4

NCORE = 2
NSUB = 16
ROWS_PER_TILE = NPAD // NSUB
CH = 128
NCHUNK = E // CH
MP_BASE = NCHUNK // NSUB
MP_EXTRA = NCHUNK % NSUB
DG_BASE = NCHUNK // (NCORE * NSUB)
DG_EXTRA = NCHUNK % (NCORE * NSUB)
STAGE_ROWS = 200
STAGE_ITERS = ROWS_PER_TILE // STAGE_ROWS
GLA = 3
NROWS = GLA + 1
NIDX = GLA + 3

_F32 = jnp.float32
_PREC = jax.lax.Precision.HIGHEST


def _sc_mesh():
    return plsc.VectorSubcoreMesh(core_axis_name="c", subcore_axis_name="s")


def _deg_body(dstd_hbm, zrow_hbm, deg_out, idxd, ones_v, stage_v, acc,
              sem_i, sem_s):
    c = lax.axis_index("c")
    s = lax.axis_index("s")
    for k in range(CH // 16):
        ones_v[pl.ds(k * 16, 16)] = jnp.ones((16,), _F32)
    nbase = s * ROWS_PER_TILE
    pltpu.sync_copy(zrow_hbm, stage_v)
    pltpu.sync_copy(stage_v, acc.at[pl.ds(nbase, ROWS_PER_TILE)])
    plsc.subcore_barrier()
    t = c * NSUB + s
    rbase = DG_BASE * t + jnp.minimum(t, DG_EXTRA)
    nch = DG_BASE + jnp.where(t < DG_EXTRA, 1, 0)

    def idx_copy(i):
        return pltpu.make_async_copy(dstd_hbm.at[rbase + i],
                                     idxd.at[lax.rem(i, 3)], sem_i)

    def sc_start(i):
        pltpu.async_copy(ones_v, acc.at[idxd.at[lax.rem(i, 3)]], sem_s,
                         add=True)

    def sc_wait(i):
        pltpu.make_async_copy(ones_v, acc.at[idxd.at[lax.rem(i, 3)]],
                              sem_s).wait()

    idx_copy(0).start()
    idx_copy(1).start()

    def step(i, carry):
        idx_copy(i).wait()

        @pl.when(i > 0)
        def _():
            sc_wait(i - 1)

        @pl.when(i + 2 < nch)
        def _():
            idx_copy(i + 2).start()

        sc_start(i)
        return carry

    lax.fori_loop(0, nch, step, 0)
    sc_wait(nch - 1)
    plsc.subcore_barrier()
    pltpu.sync_copy(acc.at[pl.ds(nbase, ROWS_PER_TILE)], stage_v)
    pltpu.sync_copy(stage_v,
                    deg_out.at[pl.ds(c * NPAD + nbase, ROWS_PER_TILE)])


def _deg_call(dstd, zrow1d):
    return pl.kernel(
        _deg_body,
        out_type=jax.ShapeDtypeStruct((NCORE * NPAD,), _F32),
        mesh=_sc_mesh(),
        scratch_types=[
            pltpu.VMEM((3, CH), jnp.int32),
            pltpu.VMEM((CH,), _F32),
            pltpu.VMEM((ROWS_PER_TILE,), _F32),
            pltpu.VMEM_SHARED((NPAD,), _F32),
            pltpu.SemaphoreType.DMA,
            pltpu.SemaphoreType.DMA,
        ],
        compiler_params=pltpu.CompilerParams(use_tc_tiling_on_sc=False),
    )(dstd, zrow1d)


def _mp_body(y0_hbm, y1_hbm, pidx_hbm, zrows_hbm, s0_out, s1_out,
             pidxv, rows_v, stage_v, acc, sem_i, sem_g, sem_s):
    c = lax.axis_index("c")
    s = lax.axis_index("s")
    nbase = s * ROWS_PER_TILE
    pltpu.sync_copy(zrows_hbm, stage_v)

    def zinit(k, carry):
        pltpu.sync_copy(stage_v, acc.at[pl.ds(nbase + k * STAGE_ROWS,
                                              STAGE_ROWS)])
        return carry

    lax.fori_loop(0, STAGE_ITERS, zinit, 0)
    plsc.subcore_barrier()
    rbase = MP_BASE * s + jnp.minimum(s, MP_EXTRA)
    nch = MP_BASE + jnp.where(s < MP_EXTRA, 1, 0)

    def run(y32, s_out):
        def idx_copy(i):
            return pltpu.make_async_copy(
                pidx_hbm.at[pl.ds(2 * (rbase + i), 2)],
                pidxv.at[lax.rem(i, NIDX)], sem_i)

        def g_copy(i):
            return pltpu.make_async_copy(
                y32.at[pidxv.at[lax.rem(i, NIDX), 0]],
                rows_v.at[lax.rem(i, NROWS)], sem_g)

        def s_start(i):
            pltpu.async_copy(rows_v.at[lax.rem(i, NROWS)],
                             acc.at[pidxv.at[lax.rem(i, NIDX), 1]], sem_s,
                             add=True)

        def s_wait(i):
            pltpu.make_async_copy(rows_v.at[lax.rem(i, NROWS)],
                                  acc.at[pidxv.at[lax.rem(i, NIDX), 1]],
                                  sem_s).wait()

        for j in range(GLA + 2):
            idx_copy(j).start()
        for j in range(GLA):
            idx_copy(j).wait()
            g_copy(j).start()

        def step(i, carry):
            @pl.when(i + GLA < nch)
            def _():
                idx_copy(i + GLA).wait()

            @pl.when(i > 0)
            def _():
                s_wait(i - 1)

            @pl.when(i + GLA + 2 < nch)
            def _():
                idx_copy(i + GLA + 2).start()

            g_copy(i).wait()

            @pl.when(i + GLA < nch)
            def _():
                g_copy(i + GLA).start()

            s_start(i)
            return carry

        lax.fori_loop(0, nch, step, 0)
        s_wait(nch - 1)
        plsc.subcore_barrier()
        out32 = s_out

        def copyout(k, carry):
            rb = nbase + k * STAGE_ROWS
            pltpu.sync_copy(acc.at[pl.ds(rb, STAGE_ROWS)], stage_v)
            pltpu.sync_copy(stage_v, out32.at[pl.ds(rb, STAGE_ROWS)])
            return carry

        lax.fori_loop(0, STAGE_ITERS, copyout, 0)

    pl.when(c == 0)(lambda: run(y0_hbm, s0_out))
    pl.when(c == 1)(lambda: run(y1_hbm, s1_out))


def _mp_call(y0, y1, pidx, zrows):
    return pl.kernel(
        _mp_body,
        out_type=[jax.ShapeDtypeStruct((NPAD, HALF), _F32),
                  jax.ShapeDtypeStruct((NPAD, HALF), _F32)],
        mesh=_sc_mesh(),
        scratch_types=[
            pltpu.VMEM((NIDX, 2, CH), jnp.int32),
            pltpu.VMEM((NROWS, CH, HALF), _F32),
            pltpu.VMEM((STAGE_ROWS, HALF), _F32),
            pltpu.VMEM_SHARED((NPAD, HALF), _F32),
            pltpu.SemaphoreType.DMA,
            pltpu.SemaphoreType.DMA,
            pltpu.SemaphoreType.DMA,
        ],
        compiler_params=pltpu.CompilerParams(use_tc_tiling_on_sc=False),
    )(y0, y1, pidx, zrows)


def _bdiag(q, nrep=4):
    r, c = q.shape
    z = jnp.zeros((r, c), _F32)
    rows = [jnp.concatenate([q if j == i else z for j in range(nrep)], axis=1)
            for i in range(nrep)]
    return jnp.concatenate(rows, axis=0)


def _rep4():
    return (lax.broadcasted_iota(jnp.int32, (4, 128), 1) // HALF ==
            lax.broadcasted_iota(jnp.int32, (4, 128), 0)).astype(_F32)


def _bias_pack(b_ref, half):
    row = b_ref[...][:, half * HALF:(half + 1) * HALF]
    return jnp.concatenate([row] * 4, axis=1)


def _dot(a, b):
    return jnp.dot(a, b, preferred_element_type=_F32, precision=_PREC)


def _prep1_body(deg4_ref, x4_ref, w1_ref, y0_ref, y1_ref, dinvp_ref):
    deg4 = deg4_ref[0] + deg4_ref[1] + 1.0
    dinvp = _dot(lax.rsqrt(deg4), _rep4())
    w1 = w1_ref[...]
    xw0p = _dot(x4_ref[...], _bdiag(w1[:, :HALF]))
    xw1p = _dot(x4_ref[...], _bdiag(w1[:, HALF:]))
    y0_ref[...] = xw0p * dinvp
    y1_ref[...] = xw1p * dinvp
    dinvp_ref[...] = dinvp


def _prep1_call(deg4, x4, w1):
    return pl.pallas_call(
        _prep1_body,
        grid=(GRID,),
        in_specs=[
            pl.BlockSpec((NCORE, NBP, 4), lambda i: (0, i, 0)),
            pl.BlockSpec((NBP, 4 * IN_CH), lambda i: (i, 0)),
            pl.BlockSpec((IN_CH, HID), lambda i: (0, 0)),
        ],
        out_specs=[
            pl.BlockSpec((NBP, 128), lambda i: (i, 0)),
            pl.BlockSpec((NBP, 128), lambda i: (i, 0)),
            pl.BlockSpec((NBP, 128), lambda i: (i, 0)),
        ],
        out_shape=[
            jax.ShapeDtypeStruct((NPK, 128), _F32),
            jax.ShapeDtypeStruct((NPK, 128), _F32),
            jax.ShapeDtypeStruct((NPK, 128), _F32),
        ],
    )(deg4, x4, w1)


def _prep2_body(s0_ref, s1_ref, y0_ref, y1_ref, dinvp_ref, w2_ref, b1_ref,
                y20_ref, y21_ref):
    dinvp = dinvp_ref[...]
    h0p = jnp.maximum((s0_ref[...] + y0_ref[...]) * dinvp
                      + _bias_pack(b1_ref, 0), 0.0)
    h1p = jnp.maximum((s1_ref[...] + y1_ref[...]) * dinvp
                      + _bias_pack(b1_ref, 1), 0.0)
    w2 = w2_ref[...]
    y20_ref[...] = (_dot(h0p, _bdiag(w2[:HALF, :HALF])) +
                    _dot(h1p, _bdiag(w2[HALF:, :HALF]))) * dinvp
    y21_ref[...] = (_dot(h0p, _bdiag(w2[:HALF, HALF:])) +
                    _dot(h1p, _bdiag(w2[HALF:, HALF:]))) * dinvp


def _prep2_call(s0, s1, y0, y1, dinvp, w2, b1):
    blk = pl.BlockSpec((NBP, 128), lambda i: (i, 0))
    return pl.pallas_call(
        _prep2_body,
        grid=(GRID,),
        in_specs=[blk, blk, blk, blk, blk,
                  pl.BlockSpec((HID, HID), lambda i: (0, 0)),
                  pl.BlockSpec((1, HID), lambda i: (0, 0))],
        out_specs=[blk, blk],
        out_shape=[
            jax.ShapeDtypeStruct((NPK, 128), _F32),
            jax.ShapeDtypeStruct((NPK, 128), _F32),
        ],
    )(s0, s1, y0, y1, dinvp, w2, b1)


def _pool_body(s0_ref, s1_ref, y20_ref, y21_ref, dinvp_ref, b2_ref,
               batch_ref, out_ref, acc_ref):
    i = pl.program_id(0)
    dinvp = dinvp_ref[...]
    h0p = (s0_ref[...] + y20_ref[...]) * dinvp + _bias_pack(b2_ref, 0)
    h1p = (s1_ref[...] + y21_ref[...]) * dinvp + _bias_pack(b2_ref, 1)
    seg0 = jnp.zeros((G, HALF), _F32)
    seg1 = jnp.zeros((G, HALF), _F32)
    counts = jnp.zeros((G, 1), _F32)
    giota = lax.broadcasted_iota(jnp.int32, (G, NBP), 0)
    b4 = batch_ref[0]
    for a in range(4):
        ohp = (giota == b4[a:a + 1, :]).astype(_F32)
        seg0 = seg0 + _dot(ohp, h0p[:, a * HALF:(a + 1) * HALF])
        seg1 = seg1 + _dot(ohp, h1p[:, a * HALF:(a + 1) * HALF])
        counts = counts + _dot(ohp, jnp.ones((NBP, 1), _F32))
    p = jnp.concatenate([seg0, seg1, counts,
                         jnp.zeros((G, HID - 1), _F32)], axis=1)

    @pl.when(i == 0)
    def _():
        acc_ref[...] = p

    @pl.when(i > 0)
    def _():
        acc_ref[...] += p

    @pl.when(i == GRID - 1)
    def _():
        a_ = acc_ref[...]
        cnt = a_[:, HID:HID + 1]
        out_ref[...] = a_[:, :HID] / jnp.maximum(cnt, 1.0)


def _pool_call(s0, s1, y20, y21, dinvp, b2, batch4):
    blk = pl.BlockSpec((NBP, 128), lambda i: (i, 0))
    return pl.pallas_call(
        _pool_body,
        grid=(GRID,),
        in_specs=[blk, blk, blk, blk, blk,
                  pl.BlockSpec((1, HID), lambda i: (0, 0)),
                  pl.BlockSpec((1, 4, NBP), lambda i: (i, 0, 0))],
        out_specs=pl.BlockSpec((G, HID), lambda i: (0, 0)),
        out_shape=jax.ShapeDtypeStruct((G, HID), _F32),
        scratch_shapes=[pltpu.VMEM((G, 2 * HID), _F32)],
    )(s0, s1, y20, y21, dinvp, b2, batch4)


@functools.partial(jax.jit)
def kernel(x, edge_index, edge_attr, batch, W1, b1, W2, b2):
    del edge_attr
    src = edge_index[0].astype(jnp.int32)
    dst = edge_index[1].astype(jnp.int32)
    x4 = jnp.pad(x, ((0, NPAD - N), (0, 0))).reshape(NPK, 4 * IN_CH)
    batch4 = jnp.pad(batch.astype(jnp.int32), (0, NPAD - N),
                     constant_values=G
                     ).reshape(GRID, NBP, 4).transpose(0, 2, 1)
    zrow1d = jnp.zeros((ROWS_PER_TILE,), _F32)
    zrows = jnp.zeros((STAGE_ROWS, HALF), _F32)
    b1r = b1.reshape(1, HID)
    b2r = b2.reshape(1, HID)
    pidx = jnp.stack([src.reshape(NCHUNK, CH), dst.reshape(NCHUNK, CH)],
                     axis=1).reshape(2 * NCHUNK, CH)
    dstd = dst.reshape(NCHUNK, CH)

    degp = _deg_call(dstd, zrow1d)
    y0, y1, dinvp = _prep1_call(degp.reshape(NCORE, NPK, 4), x4, W1)
    s0, s1 = _mp_call(y0.reshape(NPAD, HALF), y1.reshape(NPAD, HALF),
                      pidx, zrows)
    y20, y21 = _prep2_call(s0.reshape(NPK, 128), s1.reshape(NPK, 128),
                           y0, y1, dinvp, W2, b1r)
    t0, t1 = _mp_call(y20.reshape(NPAD, HALF), y21.reshape(NPAD, HALF),
                      pidx, zrows)
    return _pool_call(t0.reshape(NPK, 128), t1.reshape(NPK, 128),
                      y20, y21, dinvp, b2r, batch4)


# speedup vs baseline: 1.7139x; 1.0212x over previous
import functools

import jax
import jax.numpy as jnp
from jax import lax
from jax.experimental import pallas as pl
from jax.experimental.pallas import tpu as pltpu
from jax.experimental.pallas import tpu_sc as plsc

N = 50000
NPAD = 51200
E = 800000
IN_CH = 6
HID = 64
HALF = HID // 2
G = 64
NB = 5120
NBP = NB // 4
GRID = NPAD // NB
NPK = NPAD // 4

NCORE = 2
NSUB = 16
ROWS_PER_TILE = NPAD // NSUB
CH = 128
NCHUNK = E // CH
MP_BASE = NCHUNK // NSUB
MP_EXTRA = NCHUNK % NSUB
DG_BASE = NCHUNK // (NCORE * NSUB)
DG_EXTRA = NCHUNK % (NCORE * NSUB)
STAGE_ROWS = 80
STAGE_ITERS = ROWS_PER_TILE // STAGE_ROWS
GLA = 4
NROWS = GLA + 1
NIDX = GLA + 3

_F32 = jnp.float32
_PREC = jax.lax.Precision.HIGHEST


def _sc_mesh():
    return plsc.VectorSubcoreMesh(core_axis_name="c", subcore_axis_name="s")


def _deg_body(dstd_hbm, zrow_hbm, deg_out, idxd, ones_v, stage_v, acc,
              sem_i, sem_s):
    c = lax.axis_index("c")
    s = lax.axis_index("s")
    for k in range(CH // 16):
        ones_v[pl.ds(k * 16, 16)] = jnp.ones((16,), _F32)
    nbase = s * ROWS_PER_TILE
    pltpu.sync_copy(zrow_hbm, stage_v)
    pltpu.sync_copy(stage_v, acc.at[pl.ds(nbase, ROWS_PER_TILE)])
    plsc.subcore_barrier()
    t = c * NSUB + s
    rbase = DG_BASE * t + jnp.minimum(t, DG_EXTRA)
    nch = DG_BASE + jnp.where(t < DG_EXTRA, 1, 0)

    def idx_copy(i):
        return pltpu.make_async_copy(dstd_hbm.at[rbase + i],
                                     idxd.at[lax.rem(i, 3)], sem_i)

    def sc_start(i):
        pltpu.async_copy(ones_v, acc.at[idxd.at[lax.rem(i, 3)]], sem_s,
                         add=True)

    def sc_wait(i):
        pltpu.make_async_copy(ones_v, acc.at[idxd.at[lax.rem(i, 3)]],
                              sem_s).wait()

    idx_copy(0).start()
    idx_copy(1).start()

    def step(i, carry):
        idx_copy(i).wait()

        @pl.when(i > 0)
        def _():
            sc_wait(i - 1)

        @pl.when(i + 2 < nch)
        def _():
            idx_copy(i + 2).start()

        sc_start(i)
        return carry

    lax.fori_loop(0, nch, step, 0)
    sc_wait(nch - 1)
    plsc.subcore_barrier()
    pltpu.sync_copy(acc.at[pl.ds(nbase, ROWS_PER_TILE)], stage_v)
    pltpu.sync_copy(stage_v,
                    deg_out.at[pl.ds(c * NPAD + nbase, ROWS_PER_TILE)])


def _deg_call(dstd, zrow1d):
    return pl.kernel(
        _deg_body,
        out_type=jax.ShapeDtypeStruct((NCORE * NPAD,), _F32),
        mesh=_sc_mesh(),
        scratch_types=[
            pltpu.VMEM((3, CH), jnp.int32),
            pltpu.VMEM((CH,), _F32),
            pltpu.VMEM((ROWS_PER_TILE,), _F32),
            pltpu.VMEM_SHARED((NPAD,), _F32),
            pltpu.SemaphoreType.DMA,
            pltpu.SemaphoreType.DMA,
        ],
        compiler_params=pltpu.CompilerParams(use_tc_tiling_on_sc=False),
    )(dstd, zrow1d)


def _mp_body(y0_hbm, y1_hbm, pidx_hbm, zrows_hbm, s0_out, s1_out,
             pidxv, rows_v, stage_v, acc, sem_i, sem_g, sem_s):
    c = lax.axis_index("c")
    s = lax.axis_index("s")
    nbase = s * ROWS_PER_TILE
    pltpu.sync_copy(zrows_hbm, stage_v)

    def zinit(k, carry):
        pltpu.sync_copy(stage_v, acc.at[pl.ds(nbase + k * STAGE_ROWS,
                                              STAGE_ROWS)])
        return carry

    lax.fori_loop(0, STAGE_ITERS, zinit, 0)
    plsc.subcore_barrier()
    rbase = MP_BASE * s + jnp.minimum(s, MP_EXTRA)
    nch = MP_BASE + jnp.where(s < MP_EXTRA, 1, 0)

    def run(y32, s_out):
        def idx_copy(i):
            return pltpu.make_async_copy(
                pidx_hbm.at[pl.ds(2 * (rbase + i), 2)],
                pidxv.at[lax.rem(i, NIDX)], sem_i)

        def g_copy(i):
            return pltpu.make_async_copy(
                y32.at[pidxv.at[lax.rem(i, NIDX), 0]],
                rows_v.at[lax.rem(i, NROWS)], sem_g)

        def s_start(i):
            pltpu.async_copy(rows_v.at[lax.rem(i, NROWS)],
                             acc.at[pidxv.at[lax.rem(i, NIDX), 1]], sem_s,
                             add=True)

        def s_wait(i):
            pltpu.make_async_copy(rows_v.at[lax.rem(i, NROWS)],
                                  acc.at[pidxv.at[lax.rem(i, NIDX), 1]],
                                  sem_s).wait()

        for j in range(GLA + 2):
            idx_copy(j).start()
        for j in range(GLA):
            idx_copy(j).wait()
            g_copy(j).start()

        def step(i, carry):
            @pl.when(i + GLA < nch)
            def _():
                idx_copy(i + GLA).wait()

            @pl.when(i > 0)
            def _():
                s_wait(i - 1)

            @pl.when(i + GLA + 2 < nch)
            def _():
                idx_copy(i + GLA + 2).start()

            g_copy(i).wait()

            @pl.when(i + GLA < nch)
            def _():
                g_copy(i + GLA).start()

            s_start(i)
            return carry

        lax.fori_loop(0, nch, step, 0)
        s_wait(nch - 1)
        plsc.subcore_barrier()
        out32 = s_out

        def copyout(k, carry):
            rb = nbase + k * STAGE_ROWS
            pltpu.sync_copy(acc.at[pl.ds(rb, STAGE_ROWS)], stage_v)
            pltpu.sync_copy(stage_v, out32.at[pl.ds(rb, STAGE_ROWS)])
            return carry

        lax.fori_loop(0, STAGE_ITERS, copyout, 0)

    pl.when(c == 0)(lambda: run(y0_hbm, s0_out))
    pl.when(c == 1)(lambda: run(y1_hbm, s1_out))


def _mp_call(y0, y1, pidx, zrows):
    return pl.kernel(
        _mp_body,
        out_type=[jax.ShapeDtypeStruct((NPAD, HALF), _F32),
                  jax.ShapeDtypeStruct((NPAD, HALF), _F32)],
        mesh=_sc_mesh(),
        scratch_types=[
            pltpu.VMEM((NIDX, 2, CH), jnp.int32),
            pltpu.VMEM((NROWS, CH, HALF), _F32),
            pltpu.VMEM((STAGE_ROWS, HALF), _F32),
            pltpu.VMEM_SHARED((NPAD, HALF), _F32),
            pltpu.SemaphoreType.DMA,
            pltpu.SemaphoreType.DMA,
            pltpu.SemaphoreType.DMA,
        ],
        compiler_params=pltpu.CompilerParams(use_tc_tiling_on_sc=False),
    )(y0, y1, pidx, zrows)


def _bdiag(q, nrep=4):
    r, c = q.shape
    z = jnp.zeros((r, c), _F32)
    rows = [jnp.concatenate([q if j == i else z for j in range(nrep)], axis=1)
            for i in range(nrep)]
    return jnp.concatenate(rows, axis=0)


def _rep4():
    return (lax.broadcasted_iota(jnp.int32, (4, 128), 1) // HALF ==
            lax.broadcasted_iota(jnp.int32, (4, 128), 0)).astype(_F32)


def _bias_pack(b_ref, half):
    row = b_ref[...][:, half * HALF:(half + 1) * HALF]
    return jnp.concatenate([row] * 4, axis=1)


def _dot(a, b):
    return jnp.dot(a, b, preferred_element_type=_F32, precision=_PREC)


def _prep1_body(deg4_ref, x4_ref, w1_ref, y0_ref, y1_ref, dinvp_ref):
    deg4 = deg4_ref[0] + deg4_ref[1] + 1.0
    dinvp = _dot(lax.rsqrt(deg4), _rep4())
    w1 = w1_ref[...]
    xw0p = _dot(x4_ref[...], _bdiag(w1[:, :HALF]))
    xw1p = _dot(x4_ref[...], _bdiag(w1[:, HALF:]))
    y0_ref[...] = xw0p * dinvp
    y1_ref[...] = xw1p * dinvp
    dinvp_ref[...] = dinvp


def _prep1_call(deg4, x4, w1):
    return pl.pallas_call(
        _prep1_body,
        grid=(GRID,),
        in_specs=[
            pl.BlockSpec((NCORE, NBP, 4), lambda i: (0, i, 0)),
            pl.BlockSpec((NBP, 4 * IN_CH), lambda i: (i, 0)),
            pl.BlockSpec((IN_CH, HID), lambda i: (0, 0)),
        ],
        out_specs=[
            pl.BlockSpec((NBP, 128), lambda i: (i, 0)),
            pl.BlockSpec((NBP, 128), lambda i: (i, 0)),
            pl.BlockSpec((NBP, 128), lambda i: (i, 0)),
        ],
        out_shape=[
            jax.ShapeDtypeStruct((NPK, 128), _F32),
            jax.ShapeDtypeStruct((NPK, 128), _F32),
            jax.ShapeDtypeStruct((NPK, 128), _F32),
        ],
    )(deg4, x4, w1)


def _prep2_body(s0_ref, s1_ref, y0_ref, y1_ref, dinvp_ref, w2_ref, b1_ref,
                y20_ref, y21_ref):
    dinvp = dinvp_ref[...]
    h0p = jnp.maximum((s0_ref[...] + y0_ref[...]) * dinvp
                      + _bias_pack(b1_ref, 0), 0.0)
    h1p = jnp.maximum((s1_ref[...] + y1_ref[...]) * dinvp
                      + _bias_pack(b1_ref, 1), 0.0)
    w2 = w2_ref[...]
    y20_ref[...] = (_dot(h0p, _bdiag(w2[:HALF, :HALF])) +
                    _dot(h1p, _bdiag(w2[HALF:, :HALF]))) * dinvp
    y21_ref[...] = (_dot(h0p, _bdiag(w2[:HALF, HALF:])) +
                    _dot(h1p, _bdiag(w2[HALF:, HALF:]))) * dinvp


def _prep2_call(s0, s1, y0, y1, dinvp, w2, b1):
    blk = pl.BlockSpec((NBP, 128), lambda i: (i, 0))
    return pl.pallas_call(
        _prep2_body,
        grid=(GRID,),
        in_specs=[blk, blk, blk, blk, blk,
                  pl.BlockSpec((HID, HID), lambda i: (0, 0)),
                  pl.BlockSpec((1, HID), lambda i: (0, 0))],
        out_specs=[blk, blk],
        out_shape=[
            jax.ShapeDtypeStruct((NPK, 128), _F32),
            jax.ShapeDtypeStruct((NPK, 128), _F32),
        ],
    )(s0, s1, y0, y1, dinvp, w2, b1)


def _pool_body(s0_ref, s1_ref, y20_ref, y21_ref, dinvp_ref, b2_ref,
               batch_ref, out_ref, acc_ref):
    i = pl.program_id(0)
    dinvp = dinvp_ref[...]
    h0p = (s0_ref[...] + y20_ref[...]) * dinvp + _bias_pack(b2_ref, 0)
    h1p = (s1_ref[...] + y21_ref[...]) * dinvp + _bias_pack(b2_ref, 1)
    seg0 = jnp.zeros((G, HALF), _F32)
    seg1 = jnp.zeros((G, HALF), _F32)
    counts = jnp.zeros((G, 1), _F32)
    giota = lax.broadcasted_iota(jnp.int32, (G, NBP), 0)
    b4 = batch_ref[0]
    for a in range(4):
        ohp = (giota == b4[a:a + 1, :]).astype(_F32)
        seg0 = seg0 + _dot(ohp, h0p[:, a * HALF:(a + 1) * HALF])
        seg1 = seg1 + _dot(ohp, h1p[:, a * HALF:(a + 1) * HALF])
        counts = counts + _dot(ohp, jnp.ones((NBP, 1), _F32))
    p = jnp.concatenate([seg0, seg1, counts,
                         jnp.zeros((G, HID - 1), _F32)], axis=1)

    @pl.when(i == 0)
    def _():
        acc_ref[...] = p

    @pl.when(i > 0)
    def _():
        acc_ref[...] += p

    @pl.when(i == GRID - 1)
    def _():
        a_ = acc_ref[...]
        cnt = a_[:, HID:HID + 1]
        out_ref[...] = a_[:, :HID] / jnp.maximum(cnt, 1.0)


def _pool_call(s0, s1, y20, y21, dinvp, b2, batch4):
    blk = pl.BlockSpec((NBP, 128), lambda i: (i, 0))
    return pl.pallas_call(
        _pool_body,
        grid=(GRID,),
        in_specs=[blk, blk, blk, blk, blk,
                  pl.BlockSpec((1, HID), lambda i: (0, 0)),
                  pl.BlockSpec((1, 4, NBP), lambda i: (i, 0, 0))],
        out_specs=pl.BlockSpec((G, HID), lambda i: (0, 0)),
        out_shape=jax.ShapeDtypeStruct((G, HID), _F32),
        scratch_shapes=[pltpu.VMEM((G, 2 * HID), _F32)],
    )(s0, s1, y20, y21, dinvp, b2, batch4)


@functools.partial(jax.jit)
def kernel(x, edge_index, edge_attr, batch, W1, b1, W2, b2):
    del edge_attr
    src = edge_index[0].astype(jnp.int32)
    dst = edge_index[1].astype(jnp.int32)
    x4 = jnp.pad(x, ((0, NPAD - N), (0, 0))).reshape(NPK, 4 * IN_CH)
    batch4 = jnp.pad(batch.astype(jnp.int32), (0, NPAD - N),
                     constant_values=G
                     ).reshape(GRID, NBP, 4).transpose(0, 2, 1)
    zrow1d = jnp.zeros((ROWS_PER_TILE,), _F32)
    zrows = jnp.zeros((STAGE_ROWS, HALF), _F32)
    b1r = b1.reshape(1, HID)
    b2r = b2.reshape(1, HID)
    pidx = jnp.stack([src.reshape(NCHUNK, CH), dst.reshape(NCHUNK, CH)],
                     axis=1).reshape(2 * NCHUNK, CH)
    dstd = dst.reshape(NCHUNK, CH)

    degp = _deg_call(dstd, zrow1d)
    y0, y1, dinvp = _prep1_call(degp.reshape(NCORE, NPK, 4), x4, W1)
    s0, s1 = _mp_call(y0.reshape(NPAD, HALF), y1.reshape(NPAD, HALF),
                      pidx, zrows)
    y20, y21 = _prep2_call(s0.reshape(NPK, 128), s1.reshape(NPK, 128),
                           y0, y1, dinvp, W2, b1r)
    t0, t1 = _mp_call(y20.reshape(NPAD, HALF), y21.reshape(NPAD, HALF),
                      pidx, zrows)
    return _pool_call(t0.reshape(NPK, 128), t1.reshape(NPK, 128),
                      y20, y21, dinvp, b2r, batch4)
